# baseline (device time: 117583 ns/iter reference)
import jax
import jax.numpy as jnp
from jax import lax
from jax.experimental import pallas as pl
from jax.experimental.pallas import tpu as pltpu

C = 4
KX = 2
KY = 1
KZ = C - KX - KY


def kernel(x):
    m_sh, n = x.shape
    quarter = m_sh // 4
    ch = quarter // C

    def body(x_ref, out_ref, local_sem, sx, sy, sz, rx, rmx, ry, rz, rmy, rmz):
        my_x = lax.axis_index("x")
        my_y = lax.axis_index("y")
        my_z = lax.axis_index("z")
        xp = (1 - my_x, my_y, my_z)
        yp = (my_x, 1 - my_y, my_z)
        zp = (my_x, my_y, 1 - my_z)

        f = 2 * my_y + my_z
        fy = 2 * (1 - my_y) + my_z
        fz = 2 * my_y + (1 - my_z)
        mq = 2 * (1 - my_y) + (1 - my_z)
        own_base = my_x * m_sh
        fb = (1 - my_x) * m_sh

        barrier = pltpu.get_barrier_semaphore()
        for p in (xp, yp, zp):
            pl.semaphore_signal(
                barrier, inc=1, device_id=p,
                device_id_type=pl.DeviceIdType.MESH,
            )
        pl.semaphore_wait(barrier, 3)

        local = pltpu.make_async_copy(
            x_ref, out_ref.at[pl.ds(own_base, m_sh)], local_sem
        )
        local.start()

        def send(src, dst_off, send_sem, recv_sem, peer):
            d = pltpu.make_async_remote_copy(
                src_ref=src,
                dst_ref=out_ref.at[pl.ds(dst_off, ch)],
                send_sem=send_sem,
                recv_sem=recv_sem,
                device_id=peer,
                device_id_type=pl.DeviceIdType.MESH,
            )
            d.start()
            return d

        def wait_recv(dst_off, recv_sem):
            d = pltpu.make_async_remote_copy(
                src_ref=out_ref.at[pl.ds(dst_off, ch)],
                dst_ref=out_ref.at[pl.ds(dst_off, ch)],
                send_sem=recv_sem,
                recv_sem=recv_sem,
                device_id=xp,
                device_id_type=pl.DeviceIdType.MESH,
            )
            d.wait_recv()

        sends = []

        for c in range(C):
            off = f * quarter + c * ch
            sends.append(
                send(x_ref.at[pl.ds(off, ch)], own_base + off,
                     sx.at[c], rx.at[c], xp)
            )
        for c in range(KX):
            off = mq * quarter + c * ch
            sends.append(
                send(x_ref.at[pl.ds(off, ch)], own_base + off,
                     sx.at[C + c], rmx.at[c], xp)
            )

        for c in range(C):
            off = fb + f * quarter + c * ch
            wait_recv(off, rx.at[c])
            sends.append(send(out_ref.at[pl.ds(off, ch)], off,
                              sy.at[c], ry.at[c], yp))
            sends.append(send(out_ref.at[pl.ds(off, ch)], off,
                              sz.at[c], rz.at[c], zp))

        for i in range(KY):
            c = KX + i
            off = fb + fz * quarter + c * ch
            wait_recv(off, rz.at[c])
            sends.append(send(out_ref.at[pl.ds(off, ch)], off,
                              sy.at[C + i], rmy.at[i], yp))

        for i in range(KZ):
            c = KX + KY + i
            off = fb + fy * quarter + c * ch
            wait_recv(off, ry.at[c])
            sends.append(send(out_ref.at[pl.ds(off, ch)], off,
                              sz.at[C + i], rmz.at[i], zp))

        for c in range(C):
            if not (KX <= c < KX + KY):
                wait_recv(fb + fz * quarter + c * ch, rz.at[c])
            if not (KX + KY <= c < C):
                wait_recv(fb + fy * quarter + c * ch, ry.at[c])
        for c in range(KX):
            wait_recv(fb + mq * quarter + c * ch, rmx.at[c])
        for i in range(KY):
            wait_recv(fb + mq * quarter + (KX + i) * ch, rmy.at[i])
        for i in range(KZ):
            wait_recv(fb + mq * quarter + (KX + KY + i) * ch, rmz.at[i])

        for d in sends:
            d.wait_send()
        local.wait()

    return pl.pallas_call(
        body,
        out_shape=jax.ShapeDtypeStruct((2 * m_sh, n), x.dtype),
        in_specs=[pl.BlockSpec(memory_space=pltpu.VMEM)],
        out_specs=pl.BlockSpec(memory_space=pltpu.VMEM),
        scratch_shapes=[
            pltpu.SemaphoreType.DMA,
            pltpu.SemaphoreType.DMA((C + KX,)),
            pltpu.SemaphoreType.DMA((C + KY,)),
            pltpu.SemaphoreType.DMA((C + KZ,)),
            pltpu.SemaphoreType.DMA((C,)),
            pltpu.SemaphoreType.DMA((KX,)),
            pltpu.SemaphoreType.DMA((C,)),
            pltpu.SemaphoreType.DMA((C,)),
            pltpu.SemaphoreType.DMA((KY,)),
            pltpu.SemaphoreType.DMA((KZ,)),
        ],
        compiler_params=pltpu.CompilerParams(collective_id=0),
    )(x)


# device time: 105504 ns/iter; 1.1145x vs baseline; 1.1145x over previous
import jax
import jax.numpy as jnp
from jax import lax
from jax.experimental import pallas as pl
from jax.experimental.pallas import tpu as pltpu

C = 8
MYC = 3
MZC = 3
MXC = C - MYC - MZC


def kernel(x):
    m_sh, n = x.shape
    quarter = m_sh // 4
    ch = quarter // C

    def body(x_ref, out_ref, local_sem, sx, sy, sz, rx, rmx, ry, rz, rmy, rmz):
        my_x = lax.axis_index("x")
        my_y = lax.axis_index("y")
        my_z = lax.axis_index("z")
        xp = (1 - my_x, my_y, my_z)
        yp = (my_x, 1 - my_y, my_z)
        zp = (my_x, my_y, 1 - my_z)

        f = 2 * my_y + my_z
        fy = 2 * (1 - my_y) + my_z
        fz = 2 * my_y + (1 - my_z)
        mq = 2 * (1 - my_y) + (1 - my_z)
        own_base = my_x * m_sh
        fb = (1 - my_x) * m_sh

        barrier = pltpu.get_barrier_semaphore()
        for p in (xp, yp, zp):
            pl.semaphore_signal(
                barrier, inc=1, device_id=p,
                device_id_type=pl.DeviceIdType.MESH,
            )
        pl.semaphore_wait(barrier, 3)

        def send(src, dst_off, send_sem, recv_sem, peer):
            d = pltpu.make_async_remote_copy(
                src_ref=src,
                dst_ref=out_ref.at[pl.ds(dst_off, ch)],
                send_sem=send_sem,
                recv_sem=recv_sem,
                device_id=peer,
                device_id_type=pl.DeviceIdType.MESH,
            )
            d.start()
            return d

        def wait_recv(dst_off, recv_sem):
            d = pltpu.make_async_remote_copy(
                src_ref=out_ref.at[pl.ds(dst_off, ch)],
                dst_ref=out_ref.at[pl.ds(dst_off, ch)],
                send_sem=recv_sem,
                recv_sem=recv_sem,
                device_id=xp,
                device_id_type=pl.DeviceIdType.MESH,
            )
            d.wait_recv()

        sends = []

        for c in range(C):
            off = f * quarter + c * ch
            sends.append(
                send(x_ref.at[pl.ds(off, ch)], own_base + off,
                     sx.at[c], rx.at[c], xp)
            )
        for i in range(MXC):
            c = C - MXC + i
            off = mq * quarter + c * ch
            sends.append(
                send(x_ref.at[pl.ds(off, ch)], own_base + off,
                     sx.at[C + i], rmx.at[i], xp)
            )

        local = pltpu.make_async_copy(
            x_ref, out_ref.at[pl.ds(own_base, m_sh)], local_sem
        )
        local.start()

        for c in range(C):
            off = fb + f * quarter + c * ch
            wait_recv(off, rx.at[c])
            sends.append(send(out_ref.at[pl.ds(off, ch)], off,
                              sy.at[c], ry.at[c], yp))
            sends.append(send(out_ref.at[pl.ds(off, ch)], off,
                              sz.at[c], rz.at[c], zp))

        for i in range(MYC):
            off = fb + fz * quarter + i * ch
            wait_recv(off, rz.at[i])
            sends.append(send(out_ref.at[pl.ds(off, ch)], off,
                              sy.at[C + i], rmy.at[i], yp))

        for i in range(MZC):
            c = MYC + i
            off = fb + fy * quarter + c * ch
            wait_recv(off, ry.at[c])
            sends.append(send(out_ref.at[pl.ds(off, ch)], off,
                              sz.at[C + i], rmz.at[i], zp))

        for c in range(C):
            if not (c < MYC):
                wait_recv(fb + fz * quarter + c * ch, rz.at[c])
            if not (MYC <= c < MYC + MZC):
                wait_recv(fb + fy * quarter + c * ch, ry.at[c])
        for i in range(MYC):
            wait_recv(fb + mq * quarter + i * ch, rmy.at[i])
        for i in range(MZC):
            wait_recv(fb + mq * quarter + (MYC + i) * ch, rmz.at[i])
        for i in range(MXC):
            wait_recv(fb + mq * quarter + (C - MXC + i) * ch, rmx.at[i])

        for d in sends:
            d.wait_send()
        local.wait()

    return pl.pallas_call(
        body,
        out_shape=jax.ShapeDtypeStruct((2 * m_sh, n), x.dtype),
        in_specs=[pl.BlockSpec(memory_space=pl.ANY)],
        out_specs=pl.BlockSpec(memory_space=pl.ANY),
        scratch_shapes=[
            pltpu.SemaphoreType.DMA,
            pltpu.SemaphoreType.DMA((C + MXC,)),
            pltpu.SemaphoreType.DMA((C + MYC,)),
            pltpu.SemaphoreType.DMA((C + MZC,)),
            pltpu.SemaphoreType.DMA((C,)),
            pltpu.SemaphoreType.DMA((MXC,)),
            pltpu.SemaphoreType.DMA((C,)),
            pltpu.SemaphoreType.DMA((C,)),
            pltpu.SemaphoreType.DMA((MYC,)),
            pltpu.SemaphoreType.DMA((MZC,)),
        ],
        compiler_params=pltpu.CompilerParams(collective_id=0),
    )(x)
